# initial kernel scaffold (unmeasured)
import functools

import jax
import jax.numpy as jnp
from jax import lax
from jax.experimental import pallas as pl
from jax.experimental.pallas import tpu as pltpu

N_HEADS = 16
DH = 128
DR = 32
SCALE = (DH + DR) ** -0.5

_sem_signal = getattr(pl, "semaphore_signal", None) or pltpu.semaphore_signal
_sem_wait = getattr(pl, "semaphore_wait", None) or pltpu.semaphore_wait
_DeviceIdType = getattr(pl, "DeviceIdType", None) or pltpu.DeviceIdType
_CompilerParams = getattr(pltpu, "CompilerParams", None) or pltpu.TPUCompilerParams


def _mm(a, b):
    return lax.dot_general(
        a, b, (((1,), (0,)), ((), ())), preferred_element_type=jnp.float32
    )


def _mm_t(a, b):
    return lax.dot_general(
        a, b, (((1,), (1,)), ((), ())), preferred_element_type=jnp.float32
    )


def kernel(x, Wdkv, Wuk, Wuv, Wq, Wqr, Wkr, Wo):
    bf16 = jnp.bfloat16
    x = x.astype(bf16)
    Wdkv = Wdkv.astype(bf16)
    Wuk = Wuk.astype(bf16)
    Wuv = Wuv.astype(bf16)
    Wq = Wq.astype(bf16)
    Wqr = Wqr.astype(bf16)
    Wkr = Wkr.astype(bf16)
    Wo = Wo.astype(bf16)

    B, S, D = x.shape
    Dc_loc = Wdkv.shape[1]

    def body(
        x_ref,
        wdkv_ref,
        wuk_ref,
        wuv_ref,
        wq_ref,
        wqr_ref,
        wkr_ref,
        wo_ref,
        out_ref,
        wuk_recv,
        wuv_recv,
        c_send,
        c_recv,
        o_buf,
        send_sems,
        recv_sems,
    ):
        my_x = lax.axis_index("x")
        my_y = lax.axis_index("y")
        peer = (my_x, 1 - my_y)

        barrier = pltpu.get_barrier_semaphore()
        _sem_signal(barrier, inc=1, device_id=peer, device_id_type=_DeviceIdType.MESH)
        _sem_wait(barrier, 1)

        rdma_wuk = pltpu.make_async_remote_copy(
            src_ref=wuk_ref,
            dst_ref=wuk_recv,
            send_sem=send_sems.at[0],
            recv_sem=recv_sems.at[0],
            device_id=peer,
            device_id_type=_DeviceIdType.MESH,
        )
        rdma_wuk.start()
        rdma_wuv = pltpu.make_async_remote_copy(
            src_ref=wuv_ref,
            dst_ref=wuv_recv,
            send_sem=send_sems.at[1],
            recv_sem=recv_sems.at[1],
            device_id=peer,
            device_id_type=_DeviceIdType.MESH,
        )
        rdma_wuv.start()

        xv = x_ref[0]

        c_loc = _mm(xv, wdkv_ref[...]).astype(bf16)
        c_send[...] = c_loc
        rdma_c = pltpu.make_async_remote_copy(
            src_ref=c_send,
            dst_ref=c_recv,
            send_sem=send_sems.at[2],
            recv_sem=recv_sems.at[2],
            device_id=peer,
            device_id_type=_DeviceIdType.MESH,
        )
        rdma_c.start()

        Q = _mm(xv, wq_ref[...]).astype(bf16)
        Qr = _mm(xv, wqr_ref[...]).astype(bf16)
        Krb = _mm(xv, wkr_ref[...]).astype(bf16)
        K_part = _mm(c_loc, wuk_ref[...])
        V_part = _mm(c_loc, wuv_ref[...])

        rdma_wuk.wait()
        rdma_wuv.wait()
        rdma_c.wait()

        c_peer = c_recv[...]
        Kb = (K_part + _mm(c_peer, wuk_recv[...])).astype(bf16)
        Vb = (V_part + _mm(c_peer, wuv_recv[...])).astype(bf16)

        for h in range(N_HEADS):
            q = Q[:, h * DH : (h + 1) * DH]
            k = Kb[:, h * DH : (h + 1) * DH]
            qr = Qr[:, h * DR : (h + 1) * DR]
            s = _mm_t(q, k) + _mm_t(qr, Krb)
            s = s * SCALE
            m = jnp.max(s, axis=-1, keepdims=True)
            p = jnp.exp(s - m)
            p = p / jnp.sum(p, axis=-1, keepdims=True)
            o = _mm(p.astype(bf16), Vb[:, h * DH : (h + 1) * DH])
            o_buf[:, h * DH : (h + 1) * DH] = o.astype(bf16)

        out_ref[0] = _mm(o_buf[...], wo_ref[...])

        @functools.partial(pl.run_scoped, sem=pltpu.SemaphoreType.REGULAR)
        def _(sem):
            _sem_signal(sem, inc=1, device_id=peer, device_id_type=_DeviceIdType.MESH)
            _sem_wait(sem, 1)

    out_shape = jax.ShapeDtypeStruct((B, S, D), jnp.float32)
    return pl.pallas_call(
        body,
        out_shape=out_shape,
        in_specs=[pl.BlockSpec(memory_space=pltpu.VMEM)] * 8,
        out_specs=pl.BlockSpec(memory_space=pltpu.VMEM),
        scratch_shapes=[
            pltpu.VMEM((Dc_loc, D), bf16),
            pltpu.VMEM((Dc_loc, D), bf16),
            pltpu.VMEM((S, Dc_loc), bf16),
            pltpu.VMEM((S, Dc_loc), bf16),
            pltpu.VMEM((S, N_HEADS * DH), bf16),
            pltpu.SemaphoreType.DMA((3,)),
            pltpu.SemaphoreType.DMA((3,)),
        ],
        compiler_params=_CompilerParams(collective_id=0),
    )(x, Wdkv, Wuk, Wuv, Wq, Wqr, Wkr, Wo)


# baseline (device time: 122322 ns/iter reference)
import functools

import jax
import jax.numpy as jnp
from jax import lax
from jax.experimental import pallas as pl
from jax.experimental.pallas import tpu as pltpu

N_HEADS = 16
DH = 128
DR = 32
SCALE = (DH + DR) ** -0.5

_sem_signal = getattr(pl, "semaphore_signal", None) or pltpu.semaphore_signal
_sem_wait = getattr(pl, "semaphore_wait", None) or pltpu.semaphore_wait
_DeviceIdType = getattr(pl, "DeviceIdType", None) or pltpu.DeviceIdType
_CompilerParams = getattr(pltpu, "CompilerParams", None) or pltpu.TPUCompilerParams


def _mm(a, b):
    return lax.dot_general(
        a, b, (((1,), (0,)), ((), ())), preferred_element_type=jnp.float32
    )


def _mm_t(a, b):
    return lax.dot_general(
        a, b, (((1,), (1,)), ((), ())), preferred_element_type=jnp.float32
    )


def kernel(x, Wdkv, Wuk, Wuv, Wq, Wqr, Wkr, Wo):
    bf16 = jnp.bfloat16
    x = x.astype(bf16)
    Wdkv = Wdkv.astype(bf16)
    Wuk = Wuk.astype(bf16)
    Wuv = Wuv.astype(bf16)
    Wq = Wq.astype(bf16)
    Wqr = Wqr.astype(bf16)
    Wkr = Wkr.astype(bf16)
    Wo = Wo.astype(bf16)

    B, S, D = x.shape
    Dc_loc = Wdkv.shape[1]

    def body(
        x_ref,
        wdkv_ref,
        wuk_ref,
        wuv_ref,
        wq_ref,
        wqr_ref,
        wkr_ref,
        wo_ref,
        out_ref,
        wuk_recv,
        wuv_recv,
        c_send,
        c_recv,
        o_buf,
        send_sems,
        recv_sems,
    ):
        my_x = lax.axis_index("x")
        my_y = lax.axis_index("y")
        peer = (my_x, 1 - my_y)

        barrier = pltpu.get_barrier_semaphore()
        _sem_signal(barrier, inc=1, device_id=peer, device_id_type=_DeviceIdType.MESH)
        _sem_wait(barrier, 1)

        rdma_wuk = pltpu.make_async_remote_copy(
            src_ref=wuk_ref,
            dst_ref=wuk_recv,
            send_sem=send_sems.at[0],
            recv_sem=recv_sems.at[0],
            device_id=peer,
            device_id_type=_DeviceIdType.MESH,
        )
        rdma_wuk.start()
        rdma_wuv = pltpu.make_async_remote_copy(
            src_ref=wuv_ref,
            dst_ref=wuv_recv,
            send_sem=send_sems.at[1],
            recv_sem=recv_sems.at[1],
            device_id=peer,
            device_id_type=_DeviceIdType.MESH,
        )
        rdma_wuv.start()

        xv = x_ref[0]

        c_loc = _mm(xv, wdkv_ref[...]).astype(bf16)
        c_send[...] = c_loc
        rdma_c = pltpu.make_async_remote_copy(
            src_ref=c_send,
            dst_ref=c_recv,
            send_sem=send_sems.at[2],
            recv_sem=recv_sems.at[2],
            device_id=peer,
            device_id_type=_DeviceIdType.MESH,
        )
        rdma_c.start()

        Q = _mm(xv, wq_ref[...]).astype(bf16)
        Qr = _mm(xv, wqr_ref[...]).astype(bf16)
        Krb = _mm(xv, wkr_ref[...]).astype(bf16)
        K_part = _mm(c_loc, wuk_ref[...])
        V_part = _mm(c_loc, wuv_ref[...])

        rdma_wuk.wait()
        rdma_wuv.wait()
        rdma_c.wait()

        c_peer = c_recv[...]
        Kb = (K_part + _mm(c_peer, wuk_recv[...])).astype(bf16)
        Vb = (V_part + _mm(c_peer, wuv_recv[...])).astype(bf16)

        for h in range(N_HEADS):
            q = Q[:, h * DH : (h + 1) * DH]
            k = Kb[:, h * DH : (h + 1) * DH]
            qr = Qr[:, h * DR : (h + 1) * DR]
            s = _mm_t(q, k) + _mm_t(qr, Krb)
            s = s * SCALE
            m = jnp.max(s, axis=-1, keepdims=True)
            p = jnp.exp(s - m)
            p = p / jnp.sum(p, axis=-1, keepdims=True)
            o = _mm(p.astype(bf16), Vb[:, h * DH : (h + 1) * DH])
            o_buf[:, h * DH : (h + 1) * DH] = o.astype(bf16)

        out_ref[0] = _mm(o_buf[...], wo_ref[...])

        @functools.partial(pl.run_scoped, sem=pltpu.SemaphoreType.REGULAR)
        def _(sem):
            _sem_signal(sem, inc=1, device_id=peer, device_id_type=_DeviceIdType.MESH)
            _sem_wait(sem, 1)

    out_shape = jax.ShapeDtypeStruct((B, S, D), jnp.float32)
    return pl.pallas_call(
        body,
        out_shape=out_shape,
        in_specs=[pl.BlockSpec(memory_space=pltpu.VMEM)] * 8,
        out_specs=pl.BlockSpec(memory_space=pltpu.VMEM),
        scratch_shapes=[
            pltpu.VMEM((Dc_loc, D), bf16),
            pltpu.VMEM((Dc_loc, D), bf16),
            pltpu.VMEM((S, Dc_loc), bf16),
            pltpu.VMEM((S, Dc_loc), bf16),
            pltpu.VMEM((S, N_HEADS * DH), bf16),
            pltpu.SemaphoreType.DMA((3,)),
            pltpu.SemaphoreType.DMA((3,)),
        ],
        compiler_params=_CompilerParams(
            collective_id=0, vmem_limit_bytes=100 * 1024 * 1024
        ),
    )(x, Wdkv, Wuk, Wuv, Wq, Wqr, Wkr, Wo)


# device time: 115243 ns/iter; 1.0614x vs baseline; 1.0614x over previous
import functools

import jax
import jax.numpy as jnp
from jax import lax
from jax.experimental import pallas as pl
from jax.experimental.pallas import tpu as pltpu

N_HEADS = 16
DH = 128
DR = 32
SCALE = (DH + DR) ** -0.5
N_OUT_CHUNKS = 4

_sem_signal = getattr(pl, "semaphore_signal", None) or pltpu.semaphore_signal
_sem_wait = getattr(pl, "semaphore_wait", None) or pltpu.semaphore_wait
_DeviceIdType = getattr(pl, "DeviceIdType", None) or pltpu.DeviceIdType
_CompilerParams = getattr(pltpu, "CompilerParams", None) or pltpu.TPUCompilerParams


def _mm(a, b):
    return lax.dot_general(
        a, b, (((1,), (0,)), ((), ())), preferred_element_type=jnp.float32
    )


def _mm_t(a, b):
    return lax.dot_general(
        a, b, (((1,), (1,)), ((), ())), preferred_element_type=jnp.float32
    )


def kernel(x, Wdkv, Wuk, Wuv, Wq, Wqr, Wkr, Wo):
    bf16 = jnp.bfloat16
    x = x.astype(bf16)
    Wdkv = Wdkv.astype(bf16)
    Wuk = Wuk.astype(bf16)
    Wuv = Wuv.astype(bf16)
    Wq = Wq.astype(bf16)
    Wqr = Wqr.astype(bf16)
    Wkr = Wkr.astype(bf16)
    Wo = Wo.astype(bf16)

    B, S, D = x.shape
    Dc_loc = Wdkv.shape[1]
    S_loc = S // 2
    CH = S_loc // N_OUT_CHUNKS

    def body(
        x_ref,
        wdkv_ref,
        wuk_ref,
        wuv_ref,
        wq_ref,
        wqr_ref,
        wkr_ref,
        wo_ref,
        out_ref,
        wuk_recv,
        wuv_recv,
        c_send,
        c_recv,
        o_buf,
        out_send,
        out_recv,
        y_send_sems,
        y_recv_sems,
        x_send_sems,
        x_recv_sems,
    ):
        my_x = lax.axis_index("x")
        my_y = lax.axis_index("y")
        y_peer = (my_x, 1 - my_y)
        x_peer = (1 - my_x, my_y)

        barrier = pltpu.get_barrier_semaphore()
        _sem_signal(barrier, inc=1, device_id=y_peer, device_id_type=_DeviceIdType.MESH)
        _sem_signal(barrier, inc=1, device_id=x_peer, device_id_type=_DeviceIdType.MESH)
        _sem_wait(barrier, 2)

        rdma_wuk = pltpu.make_async_remote_copy(
            src_ref=wuk_ref,
            dst_ref=wuk_recv,
            send_sem=y_send_sems.at[0],
            recv_sem=y_recv_sems.at[0],
            device_id=y_peer,
            device_id_type=_DeviceIdType.MESH,
        )
        rdma_wuk.start()
        rdma_wuv = pltpu.make_async_remote_copy(
            src_ref=wuv_ref,
            dst_ref=wuv_recv,
            send_sem=y_send_sems.at[1],
            recv_sem=y_recv_sems.at[1],
            device_id=y_peer,
            device_id_type=_DeviceIdType.MESH,
        )
        rdma_wuv.start()

        xv = x_ref[0]

        c_loc = _mm(xv, wdkv_ref[...]).astype(bf16)
        c_send[...] = c_loc
        rdma_c = pltpu.make_async_remote_copy(
            src_ref=c_send,
            dst_ref=c_recv,
            send_sem=y_send_sems.at[2],
            recv_sem=y_recv_sems.at[2],
            device_id=y_peer,
            device_id_type=_DeviceIdType.MESH,
        )
        rdma_c.start()

        row0 = my_x * S_loc
        xq = x_ref[0, pl.ds(row0, S_loc), :]
        Q = (_mm(xq, wq_ref[...]) * SCALE).astype(bf16)
        Qr = (_mm(xq, wqr_ref[...]) * SCALE).astype(bf16)
        Krb = _mm(xv, wkr_ref[...]).astype(bf16)
        K_part = _mm(c_loc, wuk_ref[...])
        V_part = _mm(c_loc, wuv_ref[...])

        rdma_wuk.wait()
        rdma_wuv.wait()
        rdma_c.wait()

        c_peer = c_recv[...]
        Kb = (K_part + _mm(c_peer, wuk_recv[...])).astype(bf16)
        Vb = (V_part + _mm(c_peer, wuv_recv[...])).astype(bf16)

        for h in range(N_HEADS):
            q = Q[:, h * DH : (h + 1) * DH]
            k = Kb[:, h * DH : (h + 1) * DH]
            qr = Qr[:, h * DR : (h + 1) * DR]
            s = _mm_t(q, k) + _mm_t(qr, Krb)
            p = jnp.exp(s)
            denom = jnp.sum(p, axis=-1, keepdims=True)
            o = _mm(p.astype(bf16), Vb[:, h * DH : (h + 1) * DH])
            o_buf[:, h * DH : (h + 1) * DH] = (o / denom).astype(bf16)

        rdma_out = []
        for i in range(N_OUT_CHUNKS):
            o_chunk = _mm(o_buf[pl.ds(i * CH, CH), :], wo_ref[...])
            out_ref[0, pl.ds(row0 + i * CH, CH), :] = o_chunk
            out_send[pl.ds(i * CH, CH), :] = o_chunk.astype(bf16)
            rdma = pltpu.make_async_remote_copy(
                src_ref=out_send.at[pl.ds(i * CH, CH), :],
                dst_ref=out_recv.at[pl.ds(i * CH, CH), :],
                send_sem=x_send_sems.at[i],
                recv_sem=x_recv_sems.at[i],
                device_id=x_peer,
                device_id_type=_DeviceIdType.MESH,
            )
            rdma.start()
            rdma_out.append(rdma)

        peer_row0 = (1 - my_x) * S_loc
        for i, rdma in enumerate(rdma_out):
            rdma.wait_recv()
            out_ref[0, pl.ds(peer_row0 + i * CH, CH), :] = out_recv[
                pl.ds(i * CH, CH), :
            ].astype(jnp.float32)
        for rdma in rdma_out:
            rdma.wait_send()

        @functools.partial(pl.run_scoped, sem=pltpu.SemaphoreType.REGULAR)
        def _(sem):
            _sem_signal(sem, inc=1, device_id=y_peer, device_id_type=_DeviceIdType.MESH)
            _sem_signal(sem, inc=1, device_id=x_peer, device_id_type=_DeviceIdType.MESH)
            _sem_wait(sem, 2)

    out_shape = jax.ShapeDtypeStruct((B, S, D), jnp.float32)
    return pl.pallas_call(
        body,
        out_shape=out_shape,
        in_specs=[pl.BlockSpec(memory_space=pltpu.VMEM)] * 8,
        out_specs=pl.BlockSpec(memory_space=pltpu.VMEM),
        scratch_shapes=[
            pltpu.VMEM((Dc_loc, D), bf16),
            pltpu.VMEM((Dc_loc, D), bf16),
            pltpu.VMEM((S, Dc_loc), bf16),
            pltpu.VMEM((S, Dc_loc), bf16),
            pltpu.VMEM((S_loc, N_HEADS * DH), bf16),
            pltpu.VMEM((S_loc, D), bf16),
            pltpu.VMEM((S_loc, D), bf16),
            pltpu.SemaphoreType.DMA((3,)),
            pltpu.SemaphoreType.DMA((3,)),
            pltpu.SemaphoreType.DMA((N_OUT_CHUNKS,)),
            pltpu.SemaphoreType.DMA((N_OUT_CHUNKS,)),
        ],
        compiler_params=_CompilerParams(
            collective_id=0, vmem_limit_bytes=100 * 1024 * 1024
        ),
    )(x, Wdkv, Wuk, Wuv, Wq, Wqr, Wkr, Wo)


# device time: 115067 ns/iter; 1.0631x vs baseline; 1.0015x over previous
import functools

import jax
import jax.numpy as jnp
from jax import lax
from jax.experimental import pallas as pl
from jax.experimental.pallas import tpu as pltpu

N_HEADS = 16
DH = 128
DR = 32
SCALE = (DH + DR) ** -0.5
N_O_CHUNKS = 4

_sem_signal = getattr(pl, "semaphore_signal", None) or pltpu.semaphore_signal
_sem_wait = getattr(pl, "semaphore_wait", None) or pltpu.semaphore_wait
_DeviceIdType = getattr(pl, "DeviceIdType", None) or pltpu.DeviceIdType
_CompilerParams = getattr(pltpu, "CompilerParams", None) or pltpu.TPUCompilerParams


def _mm(a, b):
    return lax.dot_general(
        a, b, (((1,), (0,)), ((), ())), preferred_element_type=jnp.float32
    )


def _mm_t(a, b):
    return lax.dot_general(
        a, b, (((1,), (1,)), ((), ())), preferred_element_type=jnp.float32
    )


def kernel(x, Wdkv, Wuk, Wuv, Wq, Wqr, Wkr, Wo):
    bf16 = jnp.bfloat16
    x = x.astype(bf16)
    Wdkv = Wdkv.astype(bf16)
    Wuk = Wuk.astype(bf16)
    Wuv = Wuv.astype(bf16)
    Wq = Wq.astype(bf16)
    Wqr = Wqr.astype(bf16)
    Wkr = Wkr.astype(bf16)
    Wo = Wo.astype(bf16)

    B, S, D = x.shape
    Dc_loc = Wdkv.shape[1]
    S_loc = S // 2

    def body(
        x_ref,
        wdkv_ref,
        wuk_ref,
        wuv_ref,
        wq_ref,
        wqr_ref,
        wkr_ref,
        wo_ref,
        out_ref,
        wuk_recv,
        wuv_recv,
        c_send,
        c_recv,
        q_buf,
        qr_buf,
        kr_buf,
        k_buf,
        v_buf,
        o_mine,
        o_peer,
        y_send_sems,
        y_recv_sems,
        x_send_sems,
        x_recv_sems,
    ):
        my_x = lax.axis_index("x")
        my_y = lax.axis_index("y")
        y_peer = (my_x, 1 - my_y)
        x_peer = (1 - my_x, my_y)

        barrier = pltpu.get_barrier_semaphore()
        _sem_signal(barrier, inc=1, device_id=y_peer, device_id_type=_DeviceIdType.MESH)
        _sem_signal(barrier, inc=1, device_id=x_peer, device_id_type=_DeviceIdType.MESH)
        _sem_wait(barrier, 2)

        rdma_wuk = pltpu.make_async_remote_copy(
            src_ref=wuk_ref,
            dst_ref=wuk_recv,
            send_sem=y_send_sems.at[0],
            recv_sem=y_recv_sems.at[0],
            device_id=y_peer,
            device_id_type=_DeviceIdType.MESH,
        )
        rdma_wuk.start()
        rdma_wuv = pltpu.make_async_remote_copy(
            src_ref=wuv_ref,
            dst_ref=wuv_recv,
            send_sem=y_send_sems.at[1],
            recv_sem=y_recv_sems.at[1],
            device_id=y_peer,
            device_id_type=_DeviceIdType.MESH,
        )
        rdma_wuv.start()

        xv = x_ref[0]

        c_loc = _mm(xv, wdkv_ref[...]).astype(bf16)
        c_send[...] = c_loc
        rdma_c = pltpu.make_async_remote_copy(
            src_ref=c_send,
            dst_ref=c_recv,
            send_sem=y_send_sems.at[2],
            recv_sem=y_recv_sems.at[2],
            device_id=y_peer,
            device_id_type=_DeviceIdType.MESH,
        )
        rdma_c.start()

        row0 = my_x * S_loc
        xq = x_ref[0, pl.ds(row0, S_loc), :]
        q_buf[...] = (_mm(xq, wq_ref[...]) * SCALE).astype(bf16)
        qr_buf[...] = (_mm(xq, wqr_ref[...]) * SCALE).astype(bf16)
        kr_buf[...] = _mm(xv, wkr_ref[...]).astype(bf16)
        k_buf[...] = _mm(c_loc, wuk_ref[...]).astype(bf16)
        v_buf[...] = _mm(c_loc, wuv_ref[...]).astype(bf16)

        rdma_wuk.wait()
        rdma_wuv.wait()
        rdma_c.wait()

        c_peer = c_recv[...]
        k_buf[...] = k_buf[...] + _mm(c_peer, wuk_recv[...]).astype(bf16)
        v_buf[...] = v_buf[...] + _mm(c_peer, wuv_recv[...]).astype(bf16)

        for h in range(N_HEADS):
            q = q_buf[:, h * DH : (h + 1) * DH]
            k = k_buf[:, h * DH : (h + 1) * DH]
            qr = qr_buf[:, h * DR : (h + 1) * DR]
            s = _mm_t(q, k) + _mm_t(qr, kr_buf[...])
            p = jnp.exp(s)
            denom = jnp.sum(p, axis=-1, keepdims=True)
            o = _mm(p.astype(bf16), v_buf[:, h * DH : (h + 1) * DH])
            o_mine[:, h * DH : (h + 1) * DH] = (o / denom).astype(bf16)

        HD = N_HEADS * DH
        CW = HD // N_O_CHUNKS
        rdma_o = []
        for i in range(N_O_CHUNKS):
            rdma = pltpu.make_async_remote_copy(
                src_ref=o_mine.at[:, pl.ds(i * CW, CW)],
                dst_ref=o_peer.at[:, pl.ds(i * CW, CW)],
                send_sem=x_send_sems.at[i],
                recv_sem=x_recv_sems.at[i],
                device_id=x_peer,
                device_id_type=_DeviceIdType.MESH,
            )
            rdma.start()
            rdma_o.append(rdma)

        peer_row0 = (1 - my_x) * S_loc
        out_ref[0, pl.ds(row0, S_loc), :] = _mm(o_mine[...], wo_ref[...])
        for rdma in rdma_o:
            rdma.wait_recv()
        out_ref[0, pl.ds(peer_row0, S_loc), :] = _mm(o_peer[...], wo_ref[...])
        for rdma in rdma_o:
            rdma.wait_send()

        @functools.partial(pl.run_scoped, sem=pltpu.SemaphoreType.REGULAR)
        def _(sem):
            _sem_signal(sem, inc=1, device_id=y_peer, device_id_type=_DeviceIdType.MESH)
            _sem_signal(sem, inc=1, device_id=x_peer, device_id_type=_DeviceIdType.MESH)
            _sem_wait(sem, 2)

    out_shape = jax.ShapeDtypeStruct((B, S, D), jnp.float32)
    return pl.pallas_call(
        body,
        out_shape=out_shape,
        in_specs=[pl.BlockSpec(memory_space=pltpu.VMEM)] * 8,
        out_specs=pl.BlockSpec(memory_space=pltpu.VMEM),
        scratch_shapes=[
            pltpu.VMEM((Dc_loc, D), bf16),
            pltpu.VMEM((Dc_loc, D), bf16),
            pltpu.VMEM((S, Dc_loc), bf16),
            pltpu.VMEM((S, Dc_loc), bf16),
            pltpu.VMEM((S_loc, N_HEADS * DH), bf16),
            pltpu.VMEM((S_loc, N_HEADS * DR), bf16),
            pltpu.VMEM((S, DR), bf16),
            pltpu.VMEM((S, N_HEADS * DH), bf16),
            pltpu.VMEM((S, N_HEADS * DH), bf16),
            pltpu.VMEM((S_loc, N_HEADS * DH), bf16),
            pltpu.VMEM((S_loc, N_HEADS * DH), bf16),
            pltpu.SemaphoreType.DMA((3,)),
            pltpu.SemaphoreType.DMA((3,)),
            pltpu.SemaphoreType.DMA((N_O_CHUNKS,)),
            pltpu.SemaphoreType.DMA((N_O_CHUNKS,)),
        ],
        compiler_params=_CompilerParams(
            collective_id=0, vmem_limit_bytes=63 * 1024 * 1024
        ),
    )(x, Wdkv, Wuk, Wuv, Wq, Wqr, Wkr, Wo)


# device time: 104766 ns/iter; 1.1676x vs baseline; 1.0983x over previous
import functools

import jax
import jax.numpy as jnp
from jax import lax
from jax.experimental import pallas as pl
from jax.experimental.pallas import tpu as pltpu

N_HEADS = 16
DH = 128
DR = 32
SCALE = (DH + DR) ** -0.5
N_O_CHUNKS = 4

_sem_signal = getattr(pl, "semaphore_signal", None) or pltpu.semaphore_signal
_sem_wait = getattr(pl, "semaphore_wait", None) or pltpu.semaphore_wait
_DeviceIdType = getattr(pl, "DeviceIdType", None) or pltpu.DeviceIdType
_CompilerParams = getattr(pltpu, "CompilerParams", None) or pltpu.TPUCompilerParams


def _mm(a, b):
    return lax.dot_general(
        a, b, (((1,), (0,)), ((), ())), preferred_element_type=jnp.float32
    )


def _mm_t(a, b):
    return lax.dot_general(
        a, b, (((1,), (1,)), ((), ())), preferred_element_type=jnp.float32
    )


def kernel(x, Wdkv, Wuk, Wuv, Wq, Wqr, Wkr, Wo):
    bf16 = jnp.bfloat16
    x = x.astype(bf16)
    Wdkv = Wdkv.astype(bf16)
    Wuk = Wuk.astype(bf16)
    Wuv = Wuv.astype(bf16)
    Wq = Wq.astype(bf16)
    Wqr = Wqr.astype(bf16)
    Wkr = Wkr.astype(bf16)
    Wo = Wo.astype(bf16)

    B, S, D = x.shape
    Dc_loc = Wdkv.shape[1]
    S_loc = S // 2

    def body(
        x_ref,
        wdkv_ref,
        wuk_ref,
        wuv_ref,
        wq_ref,
        wqr_ref,
        wkr_ref,
        wo_ref,
        out_ref,
        wuk_recv,
        wuv_recv,
        c_send,
        c_recv,
        q_buf,
        qr_buf,
        kr_buf,
        k_buf,
        v_buf,
        o_mine,
        o_peer,
        y_send_sems,
        y_recv_sems,
        x_send_sems,
        x_recv_sems,
    ):
        my_x = lax.axis_index("x")
        my_y = lax.axis_index("y")
        y_peer = (my_x, 1 - my_y)
        x_peer = (1 - my_x, my_y)

        barrier = pltpu.get_barrier_semaphore()
        _sem_signal(barrier, inc=1, device_id=y_peer, device_id_type=_DeviceIdType.MESH)
        _sem_signal(barrier, inc=1, device_id=x_peer, device_id_type=_DeviceIdType.MESH)
        _sem_wait(barrier, 2)

        rdma_wuk = pltpu.make_async_remote_copy(
            src_ref=wuk_ref,
            dst_ref=wuk_recv,
            send_sem=y_send_sems.at[0],
            recv_sem=y_recv_sems.at[0],
            device_id=y_peer,
            device_id_type=_DeviceIdType.MESH,
        )
        rdma_wuk.start()
        rdma_wuv = pltpu.make_async_remote_copy(
            src_ref=wuv_ref,
            dst_ref=wuv_recv,
            send_sem=y_send_sems.at[1],
            recv_sem=y_recv_sems.at[1],
            device_id=y_peer,
            device_id_type=_DeviceIdType.MESH,
        )
        rdma_wuv.start()

        xv = x_ref[0]

        c_loc = _mm(xv, wdkv_ref[...]).astype(bf16)
        c_send[...] = c_loc
        rdma_c = pltpu.make_async_remote_copy(
            src_ref=c_send,
            dst_ref=c_recv,
            send_sem=y_send_sems.at[2],
            recv_sem=y_recv_sems.at[2],
            device_id=y_peer,
            device_id_type=_DeviceIdType.MESH,
        )
        rdma_c.start()

        row0 = my_x * S_loc
        xq = x_ref[0, pl.ds(row0, S_loc), :]
        q_buf[...] = (_mm(xq, wq_ref[...]) * SCALE).astype(bf16)
        qr_buf[...] = (_mm(xq, wqr_ref[...]) * SCALE).astype(bf16)
        kr_buf[...] = _mm(xv, wkr_ref[...]).astype(bf16)
        k_buf[...] = _mm(c_loc, wuk_ref[...]).astype(bf16)
        v_buf[...] = _mm(c_loc, wuv_ref[...]).astype(bf16)

        rdma_wuk.wait()
        rdma_wuv.wait()
        rdma_c.wait()

        c_peer = c_recv[...]
        k_buf[...] = k_buf[...] + _mm(c_peer, wuk_recv[...]).astype(bf16)
        v_buf[...] = v_buf[...] + _mm(c_peer, wuv_recv[...]).astype(bf16)

        HPC = N_HEADS // N_O_CHUNKS
        CW = HPC * DH
        rdma_o = []
        for h in range(N_HEADS):
            q = q_buf[:, h * DH : (h + 1) * DH]
            k = k_buf[:, h * DH : (h + 1) * DH]
            qr = qr_buf[:, h * DR : (h + 1) * DR]
            s = _mm_t(q, k) + _mm_t(qr, kr_buf[...])
            p = jnp.exp(s)
            denom = jnp.sum(p, axis=-1, keepdims=True)
            o = _mm(p.astype(bf16), v_buf[:, h * DH : (h + 1) * DH])
            o_mine[:, h * DH : (h + 1) * DH] = (o / denom).astype(bf16)
            if (h + 1) % HPC == 0:
                i = h // HPC
                rdma = pltpu.make_async_remote_copy(
                    src_ref=o_mine.at[:, pl.ds(i * CW, CW)],
                    dst_ref=o_peer.at[:, pl.ds(i * CW, CW)],
                    send_sem=x_send_sems.at[i],
                    recv_sem=x_recv_sems.at[i],
                    device_id=x_peer,
                    device_id_type=_DeviceIdType.MESH,
                )
                rdma.start()
                rdma_o.append(rdma)

        peer_row0 = (1 - my_x) * S_loc
        out_ref[0, pl.ds(row0, S_loc), :] = _mm(o_mine[...], wo_ref[...])
        for rdma in rdma_o:
            rdma.wait_recv()
        out_ref[0, pl.ds(peer_row0, S_loc), :] = _mm(o_peer[...], wo_ref[...])
        for rdma in rdma_o:
            rdma.wait_send()

        @functools.partial(pl.run_scoped, sem=pltpu.SemaphoreType.REGULAR)
        def _(sem):
            _sem_signal(sem, inc=1, device_id=y_peer, device_id_type=_DeviceIdType.MESH)
            _sem_signal(sem, inc=1, device_id=x_peer, device_id_type=_DeviceIdType.MESH)
            _sem_wait(sem, 2)

    out_shape = jax.ShapeDtypeStruct((B, S, D), jnp.float32)
    return pl.pallas_call(
        body,
        out_shape=out_shape,
        in_specs=[pl.BlockSpec(memory_space=pltpu.VMEM)] * 8,
        out_specs=pl.BlockSpec(memory_space=pltpu.VMEM),
        scratch_shapes=[
            pltpu.VMEM((Dc_loc, D), bf16),
            pltpu.VMEM((Dc_loc, D), bf16),
            pltpu.VMEM((S, Dc_loc), bf16),
            pltpu.VMEM((S, Dc_loc), bf16),
            pltpu.VMEM((S_loc, N_HEADS * DH), bf16),
            pltpu.VMEM((S_loc, N_HEADS * DR), bf16),
            pltpu.VMEM((S, DR), bf16),
            pltpu.VMEM((S, N_HEADS * DH), bf16),
            pltpu.VMEM((S, N_HEADS * DH), bf16),
            pltpu.VMEM((S_loc, N_HEADS * DH), bf16),
            pltpu.VMEM((S_loc, N_HEADS * DH), bf16),
            pltpu.SemaphoreType.DMA((3,)),
            pltpu.SemaphoreType.DMA((3,)),
            pltpu.SemaphoreType.DMA((N_O_CHUNKS,)),
            pltpu.SemaphoreType.DMA((N_O_CHUNKS,)),
        ],
        compiler_params=_CompilerParams(
            collective_id=0, vmem_limit_bytes=63 * 1024 * 1024
        ),
    )(x, Wdkv, Wuk, Wuv, Wq, Wqr, Wkr, Wo)


# device time: 101067 ns/iter; 1.2103x vs baseline; 1.0366x over previous
import functools

import jax
import jax.numpy as jnp
from jax import lax
from jax.experimental import pallas as pl
from jax.experimental.pallas import tpu as pltpu

N_HEADS = 16
DH = 128
DR = 32
SCALE = (DH + DR) ** -0.5
N_O_CHUNKS = 4

_sem_signal = getattr(pl, "semaphore_signal", None) or pltpu.semaphore_signal
_sem_wait = getattr(pl, "semaphore_wait", None) or pltpu.semaphore_wait
_DeviceIdType = getattr(pl, "DeviceIdType", None) or pltpu.DeviceIdType
_CompilerParams = getattr(pltpu, "CompilerParams", None) or pltpu.TPUCompilerParams


def _mm(a, b):
    return lax.dot_general(
        a, b, (((1,), (0,)), ((), ())), preferred_element_type=jnp.float32
    )


def _mm_t(a, b):
    return lax.dot_general(
        a, b, (((1,), (1,)), ((), ())), preferred_element_type=jnp.float32
    )


def kernel(x, Wdkv, Wuk, Wuv, Wq, Wqr, Wkr, Wo):
    bf16 = jnp.bfloat16
    Wdkv = Wdkv.astype(bf16)
    Wuk = Wuk.astype(bf16)
    Wuv = Wuv.astype(bf16)
    Wq = Wq.astype(bf16)
    Wqr = Wqr.astype(bf16)
    Wkr = Wkr.astype(bf16)
    Wo = Wo.astype(bf16)

    B, S, D = x.shape
    Dc_loc = Wdkv.shape[1]
    S_loc = S // 2

    def body(
        x_ref,
        wdkv_ref,
        wuk_ref,
        wuv_ref,
        wq_ref,
        wqr_ref,
        wkr_ref,
        wo_ref,
        out_ref,
        wuk_recv,
        wuv_recv,
        c_send,
        c_recv,
        x_bf,
        q_buf,
        qr_buf,
        kr_buf,
        k_buf,
        v_buf,
        o_mine,
        o_peer,
        y_send_sems,
        y_recv_sems,
        x_send_sems,
        x_recv_sems,
    ):
        my_x = lax.axis_index("x")
        my_y = lax.axis_index("y")
        y_peer = (my_x, 1 - my_y)
        x_peer = (1 - my_x, my_y)

        barrier = pltpu.get_barrier_semaphore()
        _sem_signal(barrier, inc=1, device_id=y_peer, device_id_type=_DeviceIdType.MESH)
        _sem_signal(barrier, inc=1, device_id=x_peer, device_id_type=_DeviceIdType.MESH)
        _sem_wait(barrier, 2)

        rdma_wuk = pltpu.make_async_remote_copy(
            src_ref=wuk_ref,
            dst_ref=wuk_recv,
            send_sem=y_send_sems.at[0],
            recv_sem=y_recv_sems.at[0],
            device_id=y_peer,
            device_id_type=_DeviceIdType.MESH,
        )
        rdma_wuk.start()
        rdma_wuv = pltpu.make_async_remote_copy(
            src_ref=wuv_ref,
            dst_ref=wuv_recv,
            send_sem=y_send_sems.at[1],
            recv_sem=y_recv_sems.at[1],
            device_id=y_peer,
            device_id_type=_DeviceIdType.MESH,
        )
        rdma_wuv.start()

        x_bf[...] = x_ref[0].astype(bf16)

        c_loc = _mm(x_bf[...], wdkv_ref[...]).astype(bf16)
        c_send[...] = c_loc
        rdma_c = pltpu.make_async_remote_copy(
            src_ref=c_send,
            dst_ref=c_recv,
            send_sem=y_send_sems.at[2],
            recv_sem=y_recv_sems.at[2],
            device_id=y_peer,
            device_id_type=_DeviceIdType.MESH,
        )
        rdma_c.start()

        row0 = my_x * S_loc
        xq = x_bf[pl.ds(row0, S_loc), :]
        q_buf[...] = (_mm(xq, wq_ref[...]) * SCALE).astype(bf16)
        qr_buf[...] = (_mm(xq, wqr_ref[...]) * SCALE).astype(bf16)
        kr_buf[...] = _mm(x_bf[...], wkr_ref[...]).astype(bf16)
        k_buf[...] = _mm(c_loc, wuk_ref[...]).astype(bf16)
        v_buf[...] = _mm(c_loc, wuv_ref[...]).astype(bf16)

        rdma_wuk.wait()
        rdma_wuv.wait()
        rdma_c.wait()

        c_peer = c_recv[...]
        k_buf[...] = k_buf[...] + _mm(c_peer, wuk_recv[...]).astype(bf16)
        v_buf[...] = v_buf[...] + _mm(c_peer, wuv_recv[...]).astype(bf16)

        HPC = N_HEADS // N_O_CHUNKS
        CW = HPC * DH
        rdma_o = []
        for h in range(N_HEADS):
            q = q_buf[:, h * DH : (h + 1) * DH]
            k = k_buf[:, h * DH : (h + 1) * DH]
            qr = qr_buf[:, h * DR : (h + 1) * DR]
            s = _mm_t(q, k) + _mm_t(qr, kr_buf[...])
            p = jnp.exp(s)
            denom = jnp.sum(p, axis=-1, keepdims=True)
            o = _mm(p.astype(bf16), v_buf[:, h * DH : (h + 1) * DH])
            o_mine[:, h * DH : (h + 1) * DH] = (o / denom).astype(bf16)
            if (h + 1) % HPC == 0:
                i = h // HPC
                rdma = pltpu.make_async_remote_copy(
                    src_ref=o_mine.at[:, pl.ds(i * CW, CW)],
                    dst_ref=o_peer.at[:, pl.ds(i * CW, CW)],
                    send_sem=x_send_sems.at[i],
                    recv_sem=x_recv_sems.at[i],
                    device_id=x_peer,
                    device_id_type=_DeviceIdType.MESH,
                )
                rdma.start()
                rdma_o.append(rdma)

        peer_row0 = (1 - my_x) * S_loc
        out_ref[0, pl.ds(row0, S_loc), :] = _mm(o_mine[...], wo_ref[...])
        for rdma in rdma_o:
            rdma.wait_recv()
        out_ref[0, pl.ds(peer_row0, S_loc), :] = _mm(o_peer[...], wo_ref[...])
        for rdma in rdma_o:
            rdma.wait_send()

        @functools.partial(pl.run_scoped, sem=pltpu.SemaphoreType.REGULAR)
        def _(sem):
            _sem_signal(sem, inc=1, device_id=y_peer, device_id_type=_DeviceIdType.MESH)
            _sem_signal(sem, inc=1, device_id=x_peer, device_id_type=_DeviceIdType.MESH)
            _sem_wait(sem, 2)

    out_shape = jax.ShapeDtypeStruct((B, S, D), jnp.float32)
    return pl.pallas_call(
        body,
        out_shape=out_shape,
        in_specs=[pl.BlockSpec(memory_space=pltpu.VMEM)] * 8,
        out_specs=pl.BlockSpec(memory_space=pltpu.VMEM),
        scratch_shapes=[
            pltpu.VMEM((Dc_loc, D), bf16),
            pltpu.VMEM((Dc_loc, D), bf16),
            pltpu.VMEM((S, Dc_loc), bf16),
            pltpu.VMEM((S, Dc_loc), bf16),
            pltpu.VMEM((S, D), bf16),
            pltpu.VMEM((S_loc, N_HEADS * DH), bf16),
            pltpu.VMEM((S_loc, N_HEADS * DR), bf16),
            pltpu.VMEM((S, DR), bf16),
            pltpu.VMEM((S, N_HEADS * DH), bf16),
            pltpu.VMEM((S, N_HEADS * DH), bf16),
            pltpu.VMEM((S_loc, N_HEADS * DH), bf16),
            pltpu.VMEM((S_loc, N_HEADS * DH), bf16),
            pltpu.SemaphoreType.DMA((3,)),
            pltpu.SemaphoreType.DMA((3,)),
            pltpu.SemaphoreType.DMA((N_O_CHUNKS,)),
            pltpu.SemaphoreType.DMA((N_O_CHUNKS,)),
        ],
        compiler_params=_CompilerParams(
            collective_id=0, vmem_limit_bytes=63 * 1024 * 1024
        ),
    )(x, Wdkv, Wuk, Wuv, Wq, Wqr, Wkr, Wo)


# device time: 95460 ns/iter; 1.2814x vs baseline; 1.0587x over previous
import functools

import jax
import jax.numpy as jnp
from jax import lax
from jax.experimental import pallas as pl
from jax.experimental.pallas import tpu as pltpu

N_HEADS = 16
DH = 128
DR = 32
SCALE = (DH + DR) ** -0.5
N_O_CHUNKS = 4

_sem_signal = getattr(pl, "semaphore_signal", None) or pltpu.semaphore_signal
_sem_wait = getattr(pl, "semaphore_wait", None) or pltpu.semaphore_wait
_DeviceIdType = getattr(pl, "DeviceIdType", None) or pltpu.DeviceIdType
_CompilerParams = getattr(pltpu, "CompilerParams", None) or pltpu.TPUCompilerParams


def _mm(a, b):
    return lax.dot_general(
        a, b, (((1,), (0,)), ((), ())), preferred_element_type=jnp.float32
    )


def _mm_t(a, b):
    return lax.dot_general(
        a, b, (((1,), (1,)), ((), ())), preferred_element_type=jnp.float32
    )


def kernel(x, Wdkv, Wuk, Wuv, Wq, Wqr, Wkr, Wo):
    bf16 = jnp.bfloat16
    Wq = Wq.astype(bf16)
    Wo = Wo.astype(bf16)

    B, S, D = x.shape
    Dc_loc = Wdkv.shape[1]
    S_loc = S // 2

    def body(
        x_ref,
        wdkv_ref,
        wuk_ref,
        wuv_ref,
        wq_ref,
        wqr_ref,
        wkr_ref,
        wo_ref,
        out_ref,
        wuk_send,
        wuv_send,
        wuk_recv,
        wuv_recv,
        c_send,
        c_recv,
        x_bf,
        q_buf,
        qr_buf,
        kr_buf,
        k_buf,
        v_buf,
        o_mine,
        o_peer,
        y_send_sems,
        y_recv_sems,
        x_send_sems,
        x_recv_sems,
    ):
        my_x = lax.axis_index("x")
        my_y = lax.axis_index("y")
        y_peer = (my_x, 1 - my_y)
        x_peer = (1 - my_x, my_y)

        barrier = pltpu.get_barrier_semaphore()
        _sem_signal(barrier, inc=1, device_id=y_peer, device_id_type=_DeviceIdType.MESH)
        _sem_signal(barrier, inc=1, device_id=x_peer, device_id_type=_DeviceIdType.MESH)
        _sem_wait(barrier, 2)

        wuk_send[...] = wuk_ref[...].astype(bf16)
        rdma_wuk = pltpu.make_async_remote_copy(
            src_ref=wuk_send,
            dst_ref=wuk_recv,
            send_sem=y_send_sems.at[0],
            recv_sem=y_recv_sems.at[0],
            device_id=y_peer,
            device_id_type=_DeviceIdType.MESH,
        )
        rdma_wuk.start()
        wuv_send[...] = wuv_ref[...].astype(bf16)
        rdma_wuv = pltpu.make_async_remote_copy(
            src_ref=wuv_send,
            dst_ref=wuv_recv,
            send_sem=y_send_sems.at[1],
            recv_sem=y_recv_sems.at[1],
            device_id=y_peer,
            device_id_type=_DeviceIdType.MESH,
        )
        rdma_wuv.start()

        x_bf[...] = x_ref[0].astype(bf16)

        c_loc = _mm(x_bf[...], wdkv_ref[...].astype(bf16)).astype(bf16)
        c_send[...] = c_loc
        rdma_c = pltpu.make_async_remote_copy(
            src_ref=c_send,
            dst_ref=c_recv,
            send_sem=y_send_sems.at[2],
            recv_sem=y_recv_sems.at[2],
            device_id=y_peer,
            device_id_type=_DeviceIdType.MESH,
        )
        rdma_c.start()

        row0 = my_x * S_loc
        xq = x_bf[pl.ds(row0, S_loc), :]
        q_buf[...] = (_mm(xq, wq_ref[...]) * SCALE).astype(bf16)
        qr_buf[...] = (_mm(xq, wqr_ref[...].astype(bf16)) * SCALE).astype(bf16)
        kr_buf[...] = _mm(x_bf[...], wkr_ref[...].astype(bf16)).astype(bf16)
        k_buf[...] = _mm(c_loc, wuk_send[...]).astype(bf16)
        v_buf[...] = _mm(c_loc, wuv_send[...]).astype(bf16)

        rdma_wuk.wait()
        rdma_wuv.wait()
        rdma_c.wait()

        c_peer = c_recv[...]
        k_buf[...] = k_buf[...] + _mm(c_peer, wuk_recv[...]).astype(bf16)
        v_buf[...] = v_buf[...] + _mm(c_peer, wuv_recv[...]).astype(bf16)

        HPC = N_HEADS // N_O_CHUNKS
        CW = HPC * DH
        rdma_o = []
        for h in range(N_HEADS):
            q = q_buf[:, h * DH : (h + 1) * DH]
            k = k_buf[:, h * DH : (h + 1) * DH]
            qr = qr_buf[:, h * DR : (h + 1) * DR]
            s = _mm_t(q, k) + _mm_t(qr, kr_buf[...])
            p = jnp.exp(s)
            denom = jnp.sum(p, axis=-1, keepdims=True)
            o = _mm(p.astype(bf16), v_buf[:, h * DH : (h + 1) * DH])
            o_mine[:, h * DH : (h + 1) * DH] = (o / denom).astype(bf16)
            if (h + 1) % HPC == 0:
                i = h // HPC
                rdma = pltpu.make_async_remote_copy(
                    src_ref=o_mine.at[:, pl.ds(i * CW, CW)],
                    dst_ref=o_peer.at[:, pl.ds(i * CW, CW)],
                    send_sem=x_send_sems.at[i],
                    recv_sem=x_recv_sems.at[i],
                    device_id=x_peer,
                    device_id_type=_DeviceIdType.MESH,
                )
                rdma.start()
                rdma_o.append(rdma)

        peer_row0 = (1 - my_x) * S_loc
        out_ref[0, pl.ds(row0, S_loc), :] = _mm(o_mine[...], wo_ref[...])
        for rdma in rdma_o:
            rdma.wait_recv()
        out_ref[0, pl.ds(peer_row0, S_loc), :] = _mm(o_peer[...], wo_ref[...])
        for rdma in rdma_o:
            rdma.wait_send()

        @functools.partial(pl.run_scoped, sem=pltpu.SemaphoreType.REGULAR)
        def _(sem):
            _sem_signal(sem, inc=1, device_id=y_peer, device_id_type=_DeviceIdType.MESH)
            _sem_signal(sem, inc=1, device_id=x_peer, device_id_type=_DeviceIdType.MESH)
            _sem_wait(sem, 2)

    out_shape = jax.ShapeDtypeStruct((B, S, D), jnp.float32)
    return pl.pallas_call(
        body,
        out_shape=out_shape,
        in_specs=[pl.BlockSpec(memory_space=pltpu.VMEM)] * 8,
        out_specs=pl.BlockSpec(memory_space=pltpu.VMEM),
        scratch_shapes=[
            pltpu.VMEM((Dc_loc, D), bf16),
            pltpu.VMEM((Dc_loc, D), bf16),
            pltpu.VMEM((Dc_loc, D), bf16),
            pltpu.VMEM((Dc_loc, D), bf16),
            pltpu.VMEM((S, Dc_loc), bf16),
            pltpu.VMEM((S, Dc_loc), bf16),
            pltpu.VMEM((S, D), bf16),
            pltpu.VMEM((S_loc, N_HEADS * DH), bf16),
            pltpu.VMEM((S_loc, N_HEADS * DR), bf16),
            pltpu.VMEM((S, DR), bf16),
            pltpu.VMEM((S, N_HEADS * DH), bf16),
            pltpu.VMEM((S, N_HEADS * DH), bf16),
            pltpu.VMEM((S_loc, N_HEADS * DH), bf16),
            pltpu.VMEM((S_loc, N_HEADS * DH), bf16),
            pltpu.SemaphoreType.DMA((3,)),
            pltpu.SemaphoreType.DMA((3,)),
            pltpu.SemaphoreType.DMA((N_O_CHUNKS,)),
            pltpu.SemaphoreType.DMA((N_O_CHUNKS,)),
        ],
        compiler_params=_CompilerParams(
            collective_id=0, vmem_limit_bytes=66_900_000
        ),
    )(x, Wdkv, Wuk, Wuv, Wq, Wqr, Wkr, Wo)


# device time: 75854 ns/iter; 1.6126x vs baseline; 1.2585x over previous
import functools

import jax
import jax.numpy as jnp
from jax import lax
from jax.experimental import pallas as pl
from jax.experimental.pallas import tpu as pltpu

N_HEADS = 16
DH = 128
DR = 32
SCALE = (DH + DR) ** -0.5
N_O_CHUNKS = 4
N_W_CHUNKS = 4

_sem_signal = getattr(pl, "semaphore_signal", None) or pltpu.semaphore_signal
_sem_wait = getattr(pl, "semaphore_wait", None) or pltpu.semaphore_wait
_DeviceIdType = getattr(pl, "DeviceIdType", None) or pltpu.DeviceIdType
_CompilerParams = getattr(pltpu, "CompilerParams", None) or pltpu.TPUCompilerParams


def _mm(a, b):
    return lax.dot_general(
        a, b, (((1,), (0,)), ((), ())), preferred_element_type=jnp.float32
    )


def _mm_t(a, b):
    return lax.dot_general(
        a, b, (((1,), (1,)), ((), ())), preferred_element_type=jnp.float32
    )


def kernel(x, Wdkv, Wuk, Wuv, Wq, Wqr, Wkr, Wo):
    bf16 = jnp.bfloat16

    B, S, D = x.shape
    Dc_loc = Wdkv.shape[1]
    S_loc = S // 2
    WCW = D // N_W_CHUNKS

    def body(
        x_ref,
        wdkv_ref,
        wuk_ref,
        wuv_ref,
        wq_ref,
        wqr_ref,
        wkr_ref,
        wo_ref,
        out_ref,
        wuk_send,
        wuv_send,
        wuk_recv,
        wuv_recv,
        c_send,
        c_recv,
        x_bf,
        q_buf,
        qr_buf,
        kr_buf,
        k_buf,
        v_buf,
        o_mine,
        o_peer,
        w_stage,
        y_send_sems,
        y_recv_sems,
        x_send_sems,
        x_recv_sems,
        w_sems,
    ):
        my_x = lax.axis_index("x")
        my_y = lax.axis_index("y")
        y_peer = (my_x, 1 - my_y)
        x_peer = (1 - my_x, my_y)

        def stream_weight(w_ref, consume):
            cps = []
            cp0 = pltpu.make_async_copy(
                w_ref.at[:, pl.ds(0, WCW)], w_stage.at[0], w_sems.at[0]
            )
            cp0.start()
            cps.append(cp0)
            for j in range(N_W_CHUNKS):
                if j + 1 < N_W_CHUNKS:
                    nxt = pltpu.make_async_copy(
                        w_ref.at[:, pl.ds((j + 1) * WCW, WCW)],
                        w_stage.at[(j + 1) % 2],
                        w_sems.at[(j + 1) % 2],
                    )
                    nxt.start()
                    cps.append(nxt)
                cps[j].wait()
                consume(j, w_stage[j % 2].astype(bf16))

        barrier = pltpu.get_barrier_semaphore()
        _sem_signal(barrier, inc=1, device_id=y_peer, device_id_type=_DeviceIdType.MESH)
        _sem_signal(barrier, inc=1, device_id=x_peer, device_id_type=_DeviceIdType.MESH)
        _sem_wait(barrier, 2)

        wuk_send[...] = wuk_ref[...].astype(bf16)
        rdma_wuk = pltpu.make_async_remote_copy(
            src_ref=wuk_send,
            dst_ref=wuk_recv,
            send_sem=y_send_sems.at[0],
            recv_sem=y_recv_sems.at[0],
            device_id=y_peer,
            device_id_type=_DeviceIdType.MESH,
        )
        rdma_wuk.start()
        wuv_send[...] = wuv_ref[...].astype(bf16)
        rdma_wuv = pltpu.make_async_remote_copy(
            src_ref=wuv_send,
            dst_ref=wuv_recv,
            send_sem=y_send_sems.at[1],
            recv_sem=y_recv_sems.at[1],
            device_id=y_peer,
            device_id_type=_DeviceIdType.MESH,
        )
        rdma_wuv.start()

        x_bf[...] = x_ref[0].astype(bf16)

        c_loc = _mm(x_bf[...], wdkv_ref[...].astype(bf16)).astype(bf16)
        c_send[...] = c_loc
        rdma_c = pltpu.make_async_remote_copy(
            src_ref=c_send,
            dst_ref=c_recv,
            send_sem=y_send_sems.at[2],
            recv_sem=y_recv_sems.at[2],
            device_id=y_peer,
            device_id_type=_DeviceIdType.MESH,
        )
        rdma_c.start()

        row0 = my_x * S_loc
        xq = x_bf[pl.ds(row0, S_loc), :]

        def q_chunk(j, wq_bf):
            q_buf[:, j * WCW : (j + 1) * WCW] = (_mm(xq, wq_bf) * SCALE).astype(bf16)

        stream_weight(wq_ref, q_chunk)
        qr_buf[...] = (_mm(xq, wqr_ref[...].astype(bf16)) * SCALE).astype(bf16)
        kr_buf[...] = _mm(x_bf[...], wkr_ref[...].astype(bf16)).astype(bf16)
        k_buf[...] = _mm(c_loc, wuk_send[...]).astype(bf16)
        v_buf[...] = _mm(c_loc, wuv_send[...]).astype(bf16)

        rdma_wuk.wait()
        rdma_wuv.wait()
        rdma_c.wait()

        c_peer = c_recv[...]
        k_buf[...] = k_buf[...] + _mm(c_peer, wuk_recv[...]).astype(bf16)
        v_buf[...] = v_buf[...] + _mm(c_peer, wuv_recv[...]).astype(bf16)

        HPC = N_HEADS // N_O_CHUNKS
        CW = HPC * DH
        rdma_o = []
        for h in range(N_HEADS):
            q = q_buf[:, h * DH : (h + 1) * DH]
            k = k_buf[:, h * DH : (h + 1) * DH]
            qr = qr_buf[:, h * DR : (h + 1) * DR]
            s = _mm_t(q, k) + _mm_t(qr, kr_buf[...])
            p = jnp.exp(s)
            denom = jnp.sum(p, axis=-1, keepdims=True)
            o = _mm(p.astype(bf16), v_buf[:, h * DH : (h + 1) * DH])
            o_mine[:, h * DH : (h + 1) * DH] = (o / denom).astype(bf16)
            if (h + 1) % HPC == 0:
                i = h // HPC
                rdma = pltpu.make_async_remote_copy(
                    src_ref=o_mine.at[:, pl.ds(i * CW, CW)],
                    dst_ref=o_peer.at[:, pl.ds(i * CW, CW)],
                    send_sem=x_send_sems.at[i],
                    recv_sem=x_recv_sems.at[i],
                    device_id=x_peer,
                    device_id_type=_DeviceIdType.MESH,
                )
                rdma.start()
                rdma_o.append(rdma)

        peer_row0 = (1 - my_x) * S_loc

        def out_mine_chunk(j, wo_bf):
            out_ref[0, pl.ds(row0, S_loc), pl.ds(j * WCW, WCW)] = _mm(
                o_mine[...], wo_bf
            )

        stream_weight(wo_ref, out_mine_chunk)
        for rdma in rdma_o:
            rdma.wait_recv()

        def out_peer_chunk(j, wo_bf):
            out_ref[0, pl.ds(peer_row0, S_loc), pl.ds(j * WCW, WCW)] = _mm(
                o_peer[...], wo_bf
            )

        stream_weight(wo_ref, out_peer_chunk)
        for rdma in rdma_o:
            rdma.wait_send()

        @functools.partial(pl.run_scoped, sem=pltpu.SemaphoreType.REGULAR)
        def _(sem):
            _sem_signal(sem, inc=1, device_id=y_peer, device_id_type=_DeviceIdType.MESH)
            _sem_signal(sem, inc=1, device_id=x_peer, device_id_type=_DeviceIdType.MESH)
            _sem_wait(sem, 2)

    out_shape = jax.ShapeDtypeStruct((B, S, D), jnp.float32)
    vmem = pl.BlockSpec(memory_space=pltpu.VMEM)
    hbm = pl.BlockSpec(memory_space=pl.ANY)
    return pl.pallas_call(
        body,
        out_shape=out_shape,
        in_specs=[vmem, vmem, vmem, vmem, hbm, vmem, vmem, hbm],
        out_specs=vmem,
        scratch_shapes=[
            pltpu.VMEM((Dc_loc, D), bf16),
            pltpu.VMEM((Dc_loc, D), bf16),
            pltpu.VMEM((Dc_loc, D), bf16),
            pltpu.VMEM((Dc_loc, D), bf16),
            pltpu.VMEM((S, Dc_loc), bf16),
            pltpu.VMEM((S, Dc_loc), bf16),
            pltpu.VMEM((S, D), bf16),
            pltpu.VMEM((S_loc, N_HEADS * DH), bf16),
            pltpu.VMEM((S_loc, N_HEADS * DR), bf16),
            pltpu.VMEM((S, DR), bf16),
            pltpu.VMEM((S, N_HEADS * DH), bf16),
            pltpu.VMEM((S, N_HEADS * DH), bf16),
            pltpu.VMEM((S_loc, N_HEADS * DH), bf16),
            pltpu.VMEM((S_loc, N_HEADS * DH), bf16),
            pltpu.VMEM((2, D, WCW), jnp.float32),
            pltpu.SemaphoreType.DMA((3,)),
            pltpu.SemaphoreType.DMA((3,)),
            pltpu.SemaphoreType.DMA((N_O_CHUNKS,)),
            pltpu.SemaphoreType.DMA((N_O_CHUNKS,)),
            pltpu.SemaphoreType.DMA((2,)),
        ],
        compiler_params=_CompilerParams(
            collective_id=0, vmem_limit_bytes=66_900_000
        ),
    )(x, Wdkv, Wuk, Wuv, Wq, Wqr, Wkr, Wo)
